# baseline (speedup 1.0000x reference)
import functools

import jax
import jax.numpy as jnp
from jax import lax
from jax.experimental import pallas as pl
from jax.experimental.pallas import tpu as pltpu
from jax.experimental.pallas import tpu_sc as plsc

D = 128
B = 4096 * 50
C = 64

_info = plsc.get_sparse_core_info()
NC, NS = _info.num_cores, _info.num_subcores
NW = NC * NS
B_PER_W = B // NW
N_CHUNKS = B_PER_W // C
NBUF = 10
LAG = 3


@functools.partial(
    pl.kernel,
    out_type=jax.ShapeDtypeStruct((NW, N_CHUNKS, C, D), jnp.float32),
    mesh=plsc.VectorSubcoreMesh(core_axis_name="c", subcore_axis_name="s"),
    scratch_types=[
        pltpu.VMEM((N_CHUNKS, C), jnp.int32),
        *([pltpu.VMEM((C, D), jnp.float32)] * NBUF),
        *([pltpu.SemaphoreType.DMA] * NBUF),
        *([pltpu.SemaphoreType.DMA] * NBUF),
    ],
    compiler_params=pltpu.CompilerParams(skip_device_barrier=True),
)
def _sc_gather(tok_hbm, table_hbm, out_hbm, idx_v, *rest):
    bufs = rest[:NBUF]
    gsems = rest[NBUF:2 * NBUF]
    ssems = rest[2 * NBUF:]
    wid = lax.axis_index("c") * NS + lax.axis_index("s")
    pltpu.sync_copy(tok_hbm.at[wid], idx_v)

    depth = NBUF - LAG
    for b in range(depth):
        pltpu.async_copy(table_hbm.at[idx_v.at[b]], bufs[b], gsems[b])

    def body(p, carry):
        g = p * NBUF
        for b in range(NBUF):
            j = g + b
            jn = j + depth
            bn = (b + depth) % NBUF

            @pl.when(jn < N_CHUNKS)
            def _():
                @pl.when(jn >= NBUF)
                def _():
                    pltpu.make_async_copy(
                        bufs[bn], out_hbm.at[wid, jn - NBUF],
                        ssems[bn]).wait()

                pltpu.async_copy(table_hbm.at[idx_v.at[jn]],
                                 bufs[bn], gsems[bn])

            pltpu.make_async_copy(
                table_hbm.at[idx_v.at[j]], bufs[b], gsems[b]).wait()
            pltpu.async_copy(bufs[b], out_hbm.at[wid, j], ssems[b])

        return carry

    lax.fori_loop(0, N_CHUNKS // NBUF, body, 0)

    for b in range(NBUF):
        j = N_CHUNKS - NBUF + b
        pltpu.make_async_copy(bufs[b], out_hbm.at[wid, j], ssems[b]).wait()


def kernel(recipe_tokens, embedding_table):
    toks = recipe_tokens.astype(jnp.int32).T.reshape(NW, N_CHUNKS, C)
    out = _sc_gather(toks, embedding_table)
    return out.reshape(50, 4096, D).transpose(1, 0, 2)


# speedup vs baseline: 1.0057x; 1.0057x over previous
import functools

import jax
import jax.numpy as jnp
from jax import lax
from jax.experimental import pallas as pl
from jax.experimental.pallas import tpu as pltpu
from jax.experimental.pallas import tpu_sc as plsc

D = 128
B = 4096 * 50
C = 128

_info = plsc.get_sparse_core_info()
NC, NS = _info.num_cores, _info.num_subcores
NW = NC * NS
B_PER_W = B // NW
N_CHUNKS = B_PER_W // C
NBUF = 5


@functools.partial(
    pl.kernel,
    out_type=jax.ShapeDtypeStruct((NW, N_CHUNKS, C, D), jnp.float32),
    mesh=plsc.VectorSubcoreMesh(core_axis_name="c", subcore_axis_name="s"),
    scratch_types=[
        pltpu.VMEM((N_CHUNKS, C), jnp.int32),
        *([pltpu.VMEM((C, D), jnp.float32)] * NBUF),
        *([pltpu.SemaphoreType.DMA] * NBUF),
    ],
)
def _sc_gather(tok_hbm, table_hbm, out_hbm, idx_v, *bufs_and_sems):
    bufs = bufs_and_sems[:NBUF]
    sems = bufs_and_sems[NBUF:]
    wid = lax.axis_index("c") * NS + lax.axis_index("s")
    pltpu.sync_copy(tok_hbm.at[wid], idx_v)

    for b in range(NBUF):
        pltpu.async_copy(table_hbm.at[idx_v.at[b]], bufs[b], sems[b])

    def body(p, carry):
        g = p * NBUF
        for b in range(NBUF):
            j = g + b
            pltpu.make_async_copy(
                table_hbm.at[idx_v.at[j]], bufs[b], sems[b]).wait()
            pltpu.sync_copy(bufs[b], out_hbm.at[wid, j])
            jn = j + NBUF

            @pl.when(jn < N_CHUNKS)
            def _():
                pltpu.async_copy(table_hbm.at[idx_v.at[jn]], bufs[b], sems[b])

        return carry

    lax.fori_loop(0, N_CHUNKS // NBUF, body, 0)


def kernel(recipe_tokens, embedding_table):
    toks = recipe_tokens.astype(jnp.int32).T.reshape(NW, N_CHUNKS, C)
    out = _sc_gather(toks, embedding_table)
    return out.reshape(50, 4096, D).transpose(1, 0, 2)
